# baseline (speedup 1.0000x reference)
import jax
import jax.numpy as jnp
from jax.experimental import pallas as pl
from jax.experimental.pallas import tpu as pltpu

_NCHAINS = 8


def _fused_body(q_ref, pt_ref, t_ref, att_ref, sel_ref, idx_ref, *, k, n):
    q = q_ref[...]
    pt = pt_ref[...]
    t = t_ref[0]
    sim = jax.lax.dot_general(
        q, pt, (((1,), (0,)), ((), ())),
        preferred_element_type=jnp.float32) / t

    nc = _NCHAINS
    nq = n // nc
    qq = nq // 4
    colsh = jax.lax.broadcasted_iota(
        jnp.int32, (sim.shape[0], qq), 1).astype(jnp.float32)
    v1, v2, v3, v4 = [], [], [], []
    c1, c2, c3, c4 = [], [], [], []
    for c in range(nc):
        s = [sim[:, c * nq + j * qq:c * nq + (j + 1) * qq] for j in range(4)]
        t01 = s[0] >= s[1]
        x1 = jnp.where(t01, s[0], s[1])
        x2 = jnp.where(t01, s[1], s[0])
        cx1 = jnp.where(t01, colsh, colsh + qq)
        cx2 = jnp.where(t01, colsh + qq, colsh)
        t23 = s[2] >= s[3]
        y1 = jnp.where(t23, s[2], s[3])
        y2 = jnp.where(t23, s[3], s[2])
        cy1 = jnp.where(t23, colsh + 2 * qq, colsh + 3 * qq)
        cy2 = jnp.where(t23, colsh + 3 * qq, colsh + 2 * qq)
        t1 = x1 >= y1
        v1.append(jnp.where(t1, x1, y1))
        c1.append(jnp.where(t1, cx1, cy1))
        m1 = jnp.where(t1, y1, x1)
        cm1 = jnp.where(t1, cy1, cx1)
        t2 = x2 >= y2
        v4.append(jnp.where(t2, y2, x2))
        c4.append(jnp.where(t2, cy2, cx2))
        m2 = jnp.where(t2, x2, y2)
        cm2 = jnp.where(t2, cx2, cy2)
        tm = (m1 > m2) | ((m1 == m2) & (cm1 < cm2))
        v2.append(jnp.where(tm, m1, m2))
        c2.append(jnp.where(tm, cm1, cm2))
        v3.append(jnp.where(tm, m2, m1))
        c3.append(jnp.where(tm, cm2, cm1))
    vq = [[] for _ in range(nc)]
    iq = [[] for _ in range(nc)]
    for r in range(k):
        for c in range(nc):
            m = jnp.max(v1[c], axis=1, keepdims=True)
            il = jnp.min(jnp.where(v1[c] == m, c1[c], float(nq)),
                         axis=1, keepdims=True)
            vq[c].append(m)
            iq[c].append(il + c * nq)
            if r < k - 1:
                match = c1[c] == il
                v1[c] = jnp.where(match, v2[c], v1[c])
                c1[c] = jnp.where(match, c2[c], c1[c])
                v2[c] = jnp.where(match, v3[c], v2[c])
                c2[c] = jnp.where(match, c3[c], c2[c])
                v3[c] = jnp.where(match, v4[c], v3[c])
                c3[c] = jnp.where(match, c4[c], c3[c])
                v4[c] = jnp.where(match, -jnp.inf, v4[c])
    cv = jnp.concatenate([x for vs in vq for x in vs], axis=1)
    ci = jnp.concatenate([x for ixs in iq for x in ixs], axis=1)

    vals, idxs = [], []
    for r in range(k):
        m = jnp.max(cv, axis=1, keepdims=True)
        tc = jnp.min(jnp.where(cv == m, ci, float(n)), axis=1, keepdims=True)
        vals.append(m)
        idxs.append(tc)
        if r < k - 1:
            cv = jnp.where(ci == tc, -jnp.inf, cv)
    v = jnp.concatenate(vals, axis=1)
    ix = jnp.concatenate(idxs, axis=1).astype(jnp.int32)

    e = jnp.exp(v - v[:, :1])
    inv_s = 1.0 / jnp.sum(e, axis=1, keepdims=True)
    att = jnp.where(sim >= v[:, k - 1:k],
                    jnp.exp(sim - v[:, :1]) * inv_s, 0.0)
    att_ref[...] = att
    sel_ref[...] = jax.lax.dot_general(
        att, pt, (((1,), (1,)), ((), ())),
        preferred_element_type=jnp.float32)
    idx_ref[...] = ix


def kernel(query_embedding, prompts, temperature, top_k):
    del top_k
    b, d = query_embedding.shape
    n = prompts.shape[0]
    k = min(10, n)
    pt = prompts.T

    rt = 64
    attention, selected, idx = pl.pallas_call(
        lambda qr, pr, tr, ar, sr, ir: _fused_body(
            qr, pr, tr, ar, sr, ir, k=k, n=n),
        grid=(b // rt,),
        in_specs=[
            pl.BlockSpec((rt, d), lambda i: (i, 0)),
            pl.BlockSpec((d, n), lambda i: (0, 0)),
            pl.BlockSpec(memory_space=pltpu.SMEM),
        ],
        out_specs=[
            pl.BlockSpec((rt, n), lambda i: (i, 0)),
            pl.BlockSpec((rt, d), lambda i: (i, 0)),
            pl.BlockSpec((rt, k), lambda i: (i, 0)),
        ],
        out_shape=[
            jax.ShapeDtypeStruct((b, n), jnp.float32),
            jax.ShapeDtypeStruct((b, d), jnp.float32),
            jax.ShapeDtypeStruct((b, k), jnp.int32),
        ],
        compiler_params=pltpu.CompilerParams(
            vmem_limit_bytes=100 * 1024 * 1024),
    )(query_embedding, pt, temperature)

    return (selected, attention, idx)


# speedup vs baseline: 1.1624x; 1.1624x over previous
import jax
import jax.numpy as jnp
from jax.experimental import pallas as pl
from jax.experimental.pallas import tpu as pltpu

_NCHAINS = 8


def _fused_body(q_ref, pt_ref, t_ref, att_ref, sel_ref, idx_ref, *, k, n):
    q = q_ref[...]
    pt = pt_ref[...]
    t = t_ref[0]
    sim = jax.lax.dot_general(
        q, pt, (((1,), (0,)), ((), ())),
        preferred_element_type=jnp.float32) / t

    nc = _NCHAINS
    nq = n // nc
    hq = nq // 2
    colsh = jax.lax.broadcasted_iota(
        jnp.int32, (sim.shape[0], hq), 1).astype(jnp.float32)
    vmax, vmin, cmax, cmin = [], [], [], []
    for c in range(nc):
        a = sim[:, c * nq:c * nq + hq]
        bb = sim[:, c * nq + hq:(c + 1) * nq]
        alo = a >= bb
        vmax.append(jnp.where(alo, a, bb))
        vmin.append(jnp.where(alo, bb, a))
        cmax.append(jnp.where(alo, colsh, colsh + hq))
        cmin.append(jnp.where(alo, colsh + hq, colsh))
    vq = [[] for _ in range(nc)]
    iq = [[] for _ in range(nc)]
    for r in range(k):
        for c in range(nc):
            m = jnp.max(vmax[c], axis=1, keepdims=True)
            il = jnp.min(jnp.where(vmax[c] == m, cmax[c], float(nq)),
                         axis=1, keepdims=True)
            vq[c].append(m)
            iq[c].append(il + c * nq)
            if r < k - 1:
                match = colsh == jnp.where(il >= hq, il - hq, il)
                vmax[c] = jnp.where(match, vmin[c], vmax[c])
                cmax[c] = jnp.where(match, cmin[c], cmax[c])
                vmin[c] = jnp.where(match, -jnp.inf, vmin[c])
    cv = jnp.concatenate([x for vs in vq for x in vs], axis=1)
    ci = jnp.concatenate([x for ixs in iq for x in ixs], axis=1)

    vals, idxs = [], []
    for r in range(k):
        m = jnp.max(cv, axis=1, keepdims=True)
        tc = jnp.min(jnp.where(cv == m, ci, float(n)), axis=1, keepdims=True)
        vals.append(m)
        idxs.append(tc)
        if r < k - 1:
            cv = jnp.where(ci == tc, -jnp.inf, cv)
    v = jnp.concatenate(vals, axis=1)
    ix = jnp.concatenate(idxs, axis=1).astype(jnp.int32)

    e = jnp.exp(v - v[:, :1])
    inv_s = 1.0 / jnp.sum(e, axis=1, keepdims=True)
    att = jnp.where(sim >= v[:, k - 1:k],
                    jnp.exp(sim - v[:, :1]) * inv_s, 0.0)
    att_ref[...] = att
    sel_ref[...] = jax.lax.dot_general(
        att, pt, (((1,), (1,)), ((), ())),
        preferred_element_type=jnp.float32)
    idx_ref[...] = ix


def kernel(query_embedding, prompts, temperature, top_k):
    del top_k
    b, d = query_embedding.shape
    n = prompts.shape[0]
    k = min(10, n)
    pt = prompts.T

    rt = 64
    attention, selected, idx = pl.pallas_call(
        lambda qr, pr, tr, ar, sr, ir: _fused_body(
            qr, pr, tr, ar, sr, ir, k=k, n=n),
        grid=(b // rt,),
        in_specs=[
            pl.BlockSpec((rt, d), lambda i: (i, 0)),
            pl.BlockSpec((d, n), lambda i: (0, 0)),
            pl.BlockSpec(memory_space=pltpu.SMEM),
        ],
        out_specs=[
            pl.BlockSpec((rt, n), lambda i: (i, 0)),
            pl.BlockSpec((rt, d), lambda i: (i, 0)),
            pl.BlockSpec((rt, k), lambda i: (i, 0)),
        ],
        out_shape=[
            jax.ShapeDtypeStruct((b, n), jnp.float32),
            jax.ShapeDtypeStruct((b, d), jnp.float32),
            jax.ShapeDtypeStruct((b, k), jnp.int32),
        ],
        compiler_params=pltpu.CompilerParams(
            vmem_limit_bytes=100 * 1024 * 1024),
    )(query_embedding, pt, temperature)

    return (selected, attention, idx)
